# Initial kernel scaffold; baseline (speedup 1.0000x reference)
import jax
import jax.numpy as jnp
from jax.experimental import pallas as pl


def kernel(x, adj, W0, b0, gamma0, beta0, W1, b1, gamma1, beta1, W2, b2, gamma2, beta2):
    raise NotImplementedError("write your pallas kernel here")


# speedup vs baseline: 5.3482x; 5.3482x over previous
import functools

import jax
import jax.numpy as jnp
from jax.experimental import pallas as pl
from jax.experimental.pallas import tpu as pltpu

_B, _N, _C = 2, 4096, 128
_TN = 512
_NT = _N // _TN


def _layer_body(yprev_ref, adj_ref, scale_ref, shift_ref, w_ref, b_ref,
                y_ref, sum_ref, ssq_ref, h_ref):
    bi = pl.program_id(0)
    i = pl.program_id(1)

    @pl.when(jnp.logical_and(bi == 0, i == 0))
    def _init_stats():
        sum_ref[...] = jnp.zeros_like(sum_ref)
        ssq_ref[...] = jnp.zeros_like(ssq_ref)

    @pl.when(i == 0)
    def _compute_h():
        yp = yprev_ref[0] * scale_ref[...] + shift_ref[...]
        h_ref[...] = jnp.dot(yp, w_ref[...],
                             preferred_element_type=jnp.float32)

    adj_blk = adj_ref[0]
    out = jnp.dot(adj_blk, h_ref[...], preferred_element_type=jnp.float32)

    sub = adj_ref[0, :, pl.ds(i * _TN, _TN)]
    r = jax.lax.broadcasted_iota(jnp.int32, (_TN, _TN), 0)
    c = jax.lax.broadcasted_iota(jnp.int32, (_TN, _TN), 1)
    diag = jnp.sum(jnp.where(r == c, sub, 0.0), axis=1)
    h_tile = h_ref[pl.ds(i * _TN, _TN), :]
    out = out + (1.0 - diag)[:, None] * h_tile

    y = jnp.maximum(out + b_ref[...], 0.0)
    y_ref[0] = y
    sum_ref[...] += jnp.sum(y, axis=0, keepdims=True)
    ssq_ref[...] += jnp.sum(y * y, axis=0, keepdims=True)


def _bn_body(y_ref, scale_ref, shift_ref, out_ref):
    out_ref[0] = y_ref[0] * scale_ref[...] + shift_ref[...]


_vec_spec = pl.BlockSpec((1, _C), lambda b, i: (0, 0))


def _layer(y, adj, scale, shift, w, b, interpret=False):
    return pl.pallas_call(
        _layer_body,
        grid=(_B, _NT),
        in_specs=[
            pl.BlockSpec((1, _N, _C), lambda bi, i: (bi, 0, 0)),
            pl.BlockSpec((1, _TN, _N), lambda bi, i: (bi, i, 0)),
            _vec_spec,
            _vec_spec,
            pl.BlockSpec((_C, _C), lambda bi, i: (0, 0)),
            _vec_spec,
        ],
        out_specs=[
            pl.BlockSpec((1, _TN, _C), lambda bi, i: (bi, i, 0)),
            _vec_spec,
            _vec_spec,
        ],
        out_shape=[
            jax.ShapeDtypeStruct((_B, _N, _C), jnp.float32),
            jax.ShapeDtypeStruct((1, _C), jnp.float32),
            jax.ShapeDtypeStruct((1, _C), jnp.float32),
        ],
        scratch_shapes=[pltpu.VMEM((_N, _C), jnp.float32)],
        compiler_params=pltpu.CompilerParams(
            dimension_semantics=("arbitrary", "arbitrary")),
        interpret=interpret,
    )(y, adj, scale, shift, w, b)


def _apply_bn(y, scale, shift, interpret=False):
    return pl.pallas_call(
        _bn_body,
        grid=(_B, _NT),
        in_specs=[
            pl.BlockSpec((1, _TN, _C), lambda bi, i: (bi, i, 0)),
            _vec_spec,
            _vec_spec,
        ],
        out_specs=pl.BlockSpec((1, _TN, _C), lambda bi, i: (bi, i, 0)),
        out_shape=jax.ShapeDtypeStruct((_B, _N, _C), jnp.float32),
        compiler_params=pltpu.CompilerParams(
            dimension_semantics=("arbitrary", "arbitrary")),
        interpret=interpret,
    )(y, scale, shift)


def _run(x, adj, params, interpret=False):
    n_samples = float(_B * _N)
    scale = jnp.ones((1, _C), jnp.float32)
    shift = jnp.zeros((1, _C), jnp.float32)
    y = x
    for (w, b, gamma, beta) in params:
        y, s, ss = _layer(y, adj, scale, shift, w, b.reshape(1, _C),
                          interpret=interpret)
        mean = s / n_samples
        var = ss / n_samples - mean * mean
        scale = gamma.reshape(1, _C) * jax.lax.rsqrt(var + 1e-5)
        shift = beta.reshape(1, _C) - mean * scale
    return _apply_bn(y, scale, shift, interpret=interpret)


def kernel(x, adj, W0, b0, gamma0, beta0, W1, b1, gamma1, beta1,
           W2, b2, gamma2, beta2):
    params = [(W0, b0, gamma0, beta0), (W1, b1, gamma1, beta1),
              (W2, b2, gamma2, beta2)]
    return _run(x, adj, params)


# speedup vs baseline: 5.4172x; 1.0129x over previous
import jax
import jax.numpy as jnp
from jax.experimental import pallas as pl
from jax.experimental.pallas import tpu as pltpu

_B, _N, _C = 2, 4096, 128
_TN = 512
_NT = _N // _TN


def _layer1_body(yprev_ref, adj_ref, scale_ref, shift_ref, w_ref, b_ref,
                 y_ref, sum_ref, ssq_ref, adj16_ref, h_ref):
    bi = pl.program_id(0)
    i = pl.program_id(1)

    @pl.when(jnp.logical_and(bi == 0, i == 0))
    def _init_stats():
        sum_ref[...] = jnp.zeros_like(sum_ref)
        ssq_ref[...] = jnp.zeros_like(ssq_ref)

    @pl.when(i == 0)
    def _compute_h():
        yp = yprev_ref[0] * scale_ref[...] + shift_ref[...]
        h_ref[...] = jnp.dot(yp, w_ref[...],
                             preferred_element_type=jnp.float32)

    adj_blk = adj_ref[0]
    out = jnp.dot(adj_blk, h_ref[...], preferred_element_type=jnp.float32)

    sub = adj_ref[0, :, pl.ds(i * _TN, _TN)]
    r = jax.lax.broadcasted_iota(jnp.int32, (_TN, _TN), 0)
    c = jax.lax.broadcasted_iota(jnp.int32, (_TN, _TN), 1)
    eye = r == c
    diag = jnp.sum(jnp.where(eye, sub, 0.0), axis=1)
    h_tile = h_ref[pl.ds(i * _TN, _TN), :]
    out = out + (1.0 - diag)[:, None] * h_tile

    adj16_ref[0] = adj_blk.astype(jnp.bfloat16)
    adj16_ref[0, :, pl.ds(i * _TN, _TN)] = jnp.where(
        eye, jnp.bfloat16(1.0), sub.astype(jnp.bfloat16))

    y = jnp.maximum(out + b_ref[...], 0.0)
    y_ref[0] = y
    sum_ref[...] += jnp.sum(y, axis=0, keepdims=True)
    ssq_ref[...] += jnp.sum(y * y, axis=0, keepdims=True)


def _layer_bf16_body(yprev_ref, adj16_ref, scale_ref, shift_ref, w_ref,
                     b_ref, y_ref, sum_ref, ssq_ref, h_ref):
    bi = pl.program_id(0)
    i = pl.program_id(1)

    @pl.when(jnp.logical_and(bi == 0, i == 0))
    def _init_stats():
        sum_ref[...] = jnp.zeros_like(sum_ref)
        ssq_ref[...] = jnp.zeros_like(ssq_ref)

    @pl.when(i == 0)
    def _compute_h():
        yp = yprev_ref[0] * scale_ref[...] + shift_ref[...]
        h_ref[...] = jnp.dot(yp, w_ref[...],
                             preferred_element_type=jnp.float32)

    out = jnp.dot(adj16_ref[0].astype(jnp.float32), h_ref[...],
                  preferred_element_type=jnp.float32)
    y = jnp.maximum(out + b_ref[...], 0.0)
    y_ref[0] = y
    sum_ref[...] += jnp.sum(y, axis=0, keepdims=True)
    ssq_ref[...] += jnp.sum(y * y, axis=0, keepdims=True)


def _bn_body(y_ref, scale_ref, shift_ref, out_ref):
    out_ref[0] = y_ref[0] * scale_ref[...] + shift_ref[...]


_vec_spec = pl.BlockSpec((1, _C), lambda b, i: (0, 0))
_act_spec = pl.BlockSpec((1, _N, _C), lambda bi, i: (bi, 0, 0))
_adj_spec = pl.BlockSpec((1, _TN, _N), lambda bi, i: (bi, i, 0))
_w_spec = pl.BlockSpec((_C, _C), lambda bi, i: (0, 0))
_y_spec = pl.BlockSpec((1, _TN, _C), lambda bi, i: (bi, i, 0))
_params = pltpu.CompilerParams(dimension_semantics=("arbitrary", "arbitrary"))


def _layer1(y, adj, scale, shift, w, b):
    return pl.pallas_call(
        _layer1_body,
        grid=(_B, _NT),
        in_specs=[_act_spec, _adj_spec, _vec_spec, _vec_spec, _w_spec,
                  _vec_spec],
        out_specs=[_y_spec, _vec_spec, _vec_spec, _adj_spec],
        out_shape=[
            jax.ShapeDtypeStruct((_B, _N, _C), jnp.float32),
            jax.ShapeDtypeStruct((1, _C), jnp.float32),
            jax.ShapeDtypeStruct((1, _C), jnp.float32),
            jax.ShapeDtypeStruct((_B, _N, _N), jnp.bfloat16),
        ],
        scratch_shapes=[pltpu.VMEM((_N, _C), jnp.float32)],
        compiler_params=_params,
    )(y, adj, scale, shift, w, b)


def _layer_bf16(y, adj16, scale, shift, w, b):
    return pl.pallas_call(
        _layer_bf16_body,
        grid=(_B, _NT),
        in_specs=[_act_spec, _adj_spec, _vec_spec, _vec_spec, _w_spec,
                  _vec_spec],
        out_specs=[_y_spec, _vec_spec, _vec_spec],
        out_shape=[
            jax.ShapeDtypeStruct((_B, _N, _C), jnp.float32),
            jax.ShapeDtypeStruct((1, _C), jnp.float32),
            jax.ShapeDtypeStruct((1, _C), jnp.float32),
        ],
        scratch_shapes=[pltpu.VMEM((_N, _C), jnp.float32)],
        compiler_params=_params,
    )(y, adj16, scale, shift, w, b)


def _apply_bn(y, scale, shift):
    return pl.pallas_call(
        _bn_body,
        grid=(_B, _NT),
        in_specs=[_y_spec, _vec_spec, _vec_spec],
        out_specs=_y_spec,
        out_shape=jax.ShapeDtypeStruct((_B, _N, _C), jnp.float32),
        compiler_params=_params,
    )(y, scale, shift)


def _fold_bn(s, ss, gamma, beta):
    n_samples = float(_B * _N)
    mean = s / n_samples
    var = ss / n_samples - mean * mean
    scale = gamma.reshape(1, _C) * jax.lax.rsqrt(var + 1e-5)
    shift = beta.reshape(1, _C) - mean * scale
    return scale, shift


def kernel(x, adj, W0, b0, gamma0, beta0, W1, b1, gamma1, beta1,
           W2, b2, gamma2, beta2):
    ones = jnp.ones((1, _C), jnp.float32)
    zeros = jnp.zeros((1, _C), jnp.float32)

    y, s, ss, adj16 = _layer1(x, adj, ones, zeros, W0, b0.reshape(1, _C))
    scale, shift = _fold_bn(s, ss, gamma0, beta0)

    y, s, ss = _layer_bf16(y, adj16, scale, shift, W1, b1.reshape(1, _C))
    scale, shift = _fold_bn(s, ss, gamma1, beta1)

    y, s, ss = _layer_bf16(y, adj16, scale, shift, W2, b2.reshape(1, _C))
    scale, shift = _fold_bn(s, ss, gamma2, beta2)

    return _apply_bn(y, scale, shift)


# speedup vs baseline: 5.5528x; 1.0250x over previous
import jax
import jax.numpy as jnp
from jax.experimental import pallas as pl
from jax.experimental.pallas import tpu as pltpu

_B, _N, _C = 2, 4096, 128
_TN = 512
_NT = _N // _TN
_NS = float(_B * _N)
_EPS = 1e-5


def _bn_coeffs(s, ss, gamma, beta):
    mean = s * (1.0 / _NS)
    var = ss * (1.0 / _NS) - mean * mean
    v = var + _EPS
    r = jax.lax.rsqrt(v)
    r = r * (1.5 - 0.5 * v * r * r)
    scale = gamma * r
    shift = beta - mean * scale
    return scale, shift


def _layer1_body(x_ref, adj_ref, w_ref, b_ref,
                 y_ref, sum_ref, ssq_ref, adj16_ref, h_ref):
    bi = pl.program_id(0)
    i = pl.program_id(1)

    @pl.when(jnp.logical_and(bi == 0, i == 0))
    def _init_stats():
        sum_ref[...] = jnp.zeros_like(sum_ref)
        ssq_ref[...] = jnp.zeros_like(ssq_ref)

    @pl.when(i == 0)
    def _compute_h():
        h_ref[...] = jnp.dot(x_ref[0], w_ref[...],
                             preferred_element_type=jnp.float32)

    adj_blk = adj_ref[0]
    out = jnp.dot(adj_blk, h_ref[...], preferred_element_type=jnp.float32)

    sub = adj_ref[0, :, pl.ds(i * _TN, _TN)]
    r = jax.lax.broadcasted_iota(jnp.int32, (_TN, _TN), 0)
    c = jax.lax.broadcasted_iota(jnp.int32, (_TN, _TN), 1)
    eye = r == c
    diag = jnp.sum(jnp.where(eye, sub, 0.0), axis=1)
    h_tile = h_ref[pl.ds(i * _TN, _TN), :]
    out = out + (1.0 - diag)[:, None] * h_tile

    adj16_ref[0] = adj_blk.astype(jnp.bfloat16)
    adj16_ref[0, :, pl.ds(i * _TN, _TN)] = jnp.where(
        eye, jnp.bfloat16(1.0), sub.astype(jnp.bfloat16))

    y = jnp.maximum(out + b_ref[...], 0.0)
    y_ref[0] = y
    sum_ref[...] += jnp.sum(y, axis=0, keepdims=True)
    ssq_ref[...] += jnp.sum(y * y, axis=0, keepdims=True)


def _layer_bf16_body(yprev_ref, adj16_ref, sprev_ref, ssprev_ref,
                     gamma_ref, beta_ref, w_ref, b_ref,
                     y_ref, sum_ref, ssq_ref, h2_ref):
    bi = pl.program_id(0)
    i = pl.program_id(1)

    @pl.when(jnp.logical_and(bi == 0, i == 0))
    def _init_stats():
        sum_ref[...] = jnp.zeros_like(sum_ref)
        ssq_ref[...] = jnp.zeros_like(ssq_ref)

    @pl.when(i == 0)
    def _compute_h():
        scale, shift = _bn_coeffs(sprev_ref[...], ssprev_ref[...],
                                  gamma_ref[...], beta_ref[...])
        yp = yprev_ref[0] * scale + shift
        h = jnp.dot(yp, w_ref[...], preferred_element_type=jnp.float32)
        h16 = h.astype(jnp.bfloat16)
        h2_ref[:, : _C] = h16
        h2_ref[:, _C:] = (h - h16.astype(jnp.float32)).astype(jnp.bfloat16)

    out2 = jnp.dot(adj16_ref[0], h2_ref[...],
                   preferred_element_type=jnp.float32)
    out = out2[:, : _C] + out2[:, _C:]
    y = jnp.maximum(out + b_ref[...], 0.0)
    y_ref[0] = y
    sum_ref[...] += jnp.sum(y, axis=0, keepdims=True)
    ssq_ref[...] += jnp.sum(y * y, axis=0, keepdims=True)


def _bn_body(y_ref, s_ref, ss_ref, gamma_ref, beta_ref, out_ref):
    scale, shift = _bn_coeffs(s_ref[...], ss_ref[...],
                              gamma_ref[...], beta_ref[...])
    out_ref[0] = y_ref[0] * scale + shift


_vec_spec = pl.BlockSpec((1, _C), lambda b, i: (0, 0))
_act_spec = pl.BlockSpec((1, _N, _C), lambda bi, i: (bi, 0, 0))
_adj_spec = pl.BlockSpec((1, _TN, _N), lambda bi, i: (bi, i, 0))
_w_spec = pl.BlockSpec((_C, _C), lambda bi, i: (0, 0))
_y_spec = pl.BlockSpec((1, _TN, _C), lambda bi, i: (bi, i, 0))
_params = pltpu.CompilerParams(dimension_semantics=("arbitrary", "arbitrary"))
_stat_shape = jax.ShapeDtypeStruct((1, _C), jnp.float32)


def _layer1(x, adj, w, b):
    return pl.pallas_call(
        _layer1_body,
        grid=(_B, _NT),
        in_specs=[_act_spec, _adj_spec, _w_spec, _vec_spec],
        out_specs=[_y_spec, _vec_spec, _vec_spec, _adj_spec],
        out_shape=[
            jax.ShapeDtypeStruct((_B, _N, _C), jnp.float32),
            _stat_shape,
            _stat_shape,
            jax.ShapeDtypeStruct((_B, _N, _N), jnp.bfloat16),
        ],
        scratch_shapes=[pltpu.VMEM((_N, _C), jnp.float32)],
        compiler_params=_params,
    )(x, adj, w, b)


def _layer_bf16(y, adj16, s, ss, gamma, beta, w, b):
    return pl.pallas_call(
        _layer_bf16_body,
        grid=(_B, _NT),
        in_specs=[_act_spec, _adj_spec, _vec_spec, _vec_spec, _vec_spec,
                  _vec_spec, _w_spec, _vec_spec],
        out_specs=[_y_spec, _vec_spec, _vec_spec],
        out_shape=[
            jax.ShapeDtypeStruct((_B, _N, _C), jnp.float32),
            _stat_shape,
            _stat_shape,
        ],
        scratch_shapes=[pltpu.VMEM((_N, 2 * _C), jnp.bfloat16)],
        compiler_params=_params,
    )(y, adj16, s, ss, gamma, beta, w, b)


def _apply_bn(y, s, ss, gamma, beta):
    return pl.pallas_call(
        _bn_body,
        grid=(_B, _NT),
        in_specs=[_y_spec, _vec_spec, _vec_spec, _vec_spec, _vec_spec],
        out_specs=_y_spec,
        out_shape=jax.ShapeDtypeStruct((_B, _N, _C), jnp.float32),
        compiler_params=_params,
    )(y, s, ss, gamma, beta)


def kernel(x, adj, W0, b0, gamma0, beta0, W1, b1, gamma1, beta1,
           W2, b2, gamma2, beta2):
    r = lambda v: v.reshape(1, _C)

    y, s, ss, adj16 = _layer1(x, adj, W0, r(b0))
    y, s, ss = _layer_bf16(y, adj16, s, ss, r(gamma0), r(beta0), W1, r(b1))
    y, s, ss = _layer_bf16(y, adj16, s, ss, r(gamma1), r(beta1), W2, r(b2))
    return _apply_bn(y, s, ss, r(gamma2), r(beta2))
